# baseline (speedup 1.0000x reference)
import functools

import jax
import jax.numpy as jnp
from jax import lax
from jax.experimental import pallas as pl
from jax.experimental.pallas import tpu as pltpu
from jax.experimental.pallas import tpu_sc as plsc

Q = 64
K = 16
BLK = 8192
CH = 128
R = 3
NCHAIN = 4
BIG_I = 2**30


def _ray_dirs(ro_ref, rd_ref):
    ox = ro_ref[:, 0:1]
    oy = ro_ref[:, 1:2]
    oz = ro_ref[:, 2:3]
    rdx = rd_ref[:, 0:1]
    rdy = rd_ref[:, 1:2]
    rdz = rd_ref[:, 2:3]
    inv = 1.0 / (jnp.sqrt(rdx * rdx + rdy * rdy + rdz * rdz) + 1e-12)
    return ox, oy, oz, rdx * inv, rdy * inv, rdz * inv


def _ray_slices(ro_ref, rd_ref, s):
    ox = ro_ref[s:s + 8, 0:1]
    oy = ro_ref[s:s + 8, 1:2]
    oz = ro_ref[s:s + 8, 2:3]
    rdx = rd_ref[s:s + 8, 0:1]
    rdy = rd_ref[s:s + 8, 1:2]
    rdz = rd_ref[s:s + 8, 2:3]
    inv = 1.0 / (jnp.sqrt(rdx * rdx + rdy * rdy + rdz * rdz) + 1e-12)
    return ox, oy, oz, rdx * inv, rdy * inv, rdz * inv


def _topk_body(ro_ref, rd_ref, pts_ref, cv_ref, ci_ref):
    for half in range(Q // 8 // NCHAIN):
        rgs = [half * NCHAIN + t for t in range(NCHAIN)]
        rays = [_ray_slices(ro_ref, rd_ref, rg * 8) for rg in rgs]
        mv = [[jnp.full((8, CH), jnp.inf, jnp.float32) for _ in range(R)]
              for _ in rgs]
        mi = [[jnp.full((8, CH), BIG_I, jnp.int32) for _ in range(R)]
              for _ in rgs]

        for c in range(BLK // CH):
            px = jnp.broadcast_to(pts_ref[0:1, c * CH:(c + 1) * CH], (8, CH))
            py = jnp.broadcast_to(pts_ref[1:2, c * CH:(c + 1) * CH], (8, CH))
            pz = jnp.broadcast_to(pts_ref[2:3, c * CH:(c + 1) * CH], (8, CH))
            for t in range(NCHAIN):
                ox, oy, oz, dx, dy, dz = rays[t]
                xs = px - ox
                ys = py - oy
                zs = pz - oz
                walk = xs * dx + ys * dy + zs * dz
                qx = xs - walk * dx
                qy = ys - walk * dy
                qz = zs - walk * dz
                x = qx * qx + qy * qy + qz * qz
                m0, m1, m2 = mv[t]
                i0, i1, i2 = mi[t]
                b0 = x < m0
                b1 = x < m1
                b2 = x < m2
                mv[t] = [jnp.minimum(x, m0),
                         jnp.minimum(jnp.maximum(x, m0), m1),
                         jnp.minimum(jnp.maximum(x, m1), m2)]
                mi[t] = [jnp.where(b0, c, i0),
                         jnp.where(b0, i0, jnp.where(b1, c, i1)),
                         jnp.where(b1, i1, jnp.where(b2, c, i2))]

        for t in range(NCHAIN):
            s = rgs[t] * 8
            for r in range(R):
                cv_ref[0, s:s + 8, r * CH:(r + 1) * CH] = mv[t][r]
                ci_ref[0, s:s + 8, r * CH:(r + 1) * CH] = mi[t][r]


def _extract_body(nb_s, cv_ref, ci_ref, topi_ref, topd2_ref):
    lane = jnp.bitwise_and(
        lax.broadcasted_iota(jnp.int32, (Q, R * CH), 1), CH - 1)
    vals = jnp.concatenate([cv_ref[b] for b in range(nb_s)], axis=1)
    idxs = jnp.concatenate(
        [b * BLK + ci_ref[b] * CH + lane for b in range(nb_s)], axis=1)
    for k in range(K):
        minv = jnp.min(vals, axis=1, keepdims=True)
        cidx = jnp.where(vals == minv, idxs, BIG_I)
        pick = jnp.min(cidx, axis=1, keepdims=True)
        topd2_ref[:, k:k + 1] = minv
        topi_ref[:, k:k + 1] = pick
        vals = jnp.where(cidx == pick, jnp.inf, vals)


def _finish_body(ro_ref, rd_ref, d2_ref, ti_ref, rx_ref, ry_ref, rz_ref,
                 gx_ref, gy_ref, gz_ref, dist_ref, walk_ref, az_ref,
                 pitch_ref):
    ox, oy, oz, dx, dy, dz = _ray_dirs(ro_ref, rd_ref)
    lane = jnp.bitwise_and(
        lax.broadcasted_iota(jnp.int32, (Q, K * CH), 1), CH - 1)
    tl = jnp.bitwise_and(ti_ref[...], CH - 1)
    tlexp = jnp.broadcast_to(tl[:, :, None], (Q, K, CH)).reshape(Q, K * CH)
    mask = lane == tlexp

    def pick(rref):
        m = jnp.where(mask, rref[...], 0.0)
        return m.reshape(Q, K, CH).sum(axis=-1)

    gx = pick(rx_ref)
    gy = pick(ry_ref)
    gz = pick(rz_ref)
    gx_ref[...] = gx
    gy_ref[...] = gy
    gz_ref[...] = gz
    dist_ref[...] = jnp.sqrt(d2_ref[...] + 1e-12)
    vx = gx - ox
    vy = gy - oy
    vz = gz - oz
    walk_ref[...] = vx * dx + vy * dy + vz * dz
    vn = jnp.sqrt(vx * vx + vy * vy + vz * vz) + 1e-12
    az_ref[...] = jnp.arctan2(vy, vx)
    ct = jnp.clip(vz / vn, -1.0 + 1e-6, 1.0 - 1e-6)
    pitch_ref[...] = jnp.arctan2(jnp.sqrt((1.0 - ct) * (1.0 + ct)), ct)


def _make_sc_rowgather(plane_stride):
    mesh = plsc.VectorSubcoreMesh(core_axis_name="c", subcore_axis_name="s")
    info = plsc.get_sparse_core_info()
    nw = info.num_cores * info.num_subcores
    per_w = (Q * K) // nw
    qk = Q * K

    @functools.partial(
        pl.kernel, mesh=mesh,
        compiler_params=pltpu.CompilerParams(use_tc_tiling_on_sc=False),
        out_type=jax.ShapeDtypeStruct((3 * qk, 128), jnp.float32),
        scratch_types=[
            pltpu.VMEM((per_w,), jnp.int32),
            pltpu.VMEM((3 * per_w,), jnp.int32),
            pltpu.VMEM((3 * per_w, 128), jnp.float32),
            pltpu.SemaphoreType.DMA,
        ],
    )
    def gather_k(table_hbm, idx_hbm, out_hbm, idx_v, tr_v, rows_v, sem):
        wid = lax.axis_index("s") * info.num_cores + lax.axis_index("c")
        base = wid * per_w
        pltpu.sync_copy(idx_hbm.at[pl.ds(base, per_w)], idx_v)
        for h in range(per_w // 16):
            v = idx_v[pl.ds(h * 16, 16)]
            xr = lax.shift_right_logical(v, 7)
            tr_v[pl.ds(h * 16, 16)] = xr
            tr_v[pl.ds(per_w + h * 16, 16)] = xr + plane_stride
            tr_v[pl.ds(2 * per_w + h * 16, 16)] = xr + 2 * plane_stride
        pltpu.async_copy(table_hbm.at[tr_v], rows_v, sem).wait()
        for p in range(3):
            pltpu.sync_copy(rows_v.at[pl.ds(p * per_w, per_w)],
                            out_hbm.at[pl.ds(p * qk + base, per_w)])

    return gather_k


def _sc_gather(ptsT, idx_flat):
    plane_stride = ptsT.shape[1] // 128
    tab = ptsT.reshape(-1, 128)
    return _make_sc_rowgather(plane_stride)(tab, idx_flat)


def _topk_call(ray_o, ray_d, points):
    n = points.shape[0]
    nb = (n + BLK - 1) // BLK
    npad = nb * BLK
    ptsT = jnp.pad(points, ((0, npad - n), (0, 0)),
                   constant_values=1e18).T

    cv, ci = pl.pallas_call(
        _topk_body,
        grid=(nb,),
        in_specs=[
            pl.BlockSpec((Q, 3), lambda i: (0, 0)),
            pl.BlockSpec((Q, 3), lambda i: (0, 0)),
            pl.BlockSpec((3, BLK), lambda i: (0, i)),
        ],
        out_specs=[pl.BlockSpec((1, Q, R * CH), lambda i: (i, 0, 0))] * 2,
        out_shape=(
            jax.ShapeDtypeStruct((nb, Q, R * CH), jnp.float32),
            jax.ShapeDtypeStruct((nb, Q, R * CH), jnp.int32),
        ),
        compiler_params=pltpu.CompilerParams(
            dimension_semantics=("arbitrary",)),
    )(ray_o, ray_d, ptsT)

    return pl.pallas_call(
        functools.partial(_extract_body, nb),
        in_specs=[
            pl.BlockSpec((nb, Q, R * CH), lambda: (0, 0, 0)),
            pl.BlockSpec((nb, Q, R * CH), lambda: (0, 0, 0)),
        ],
        out_specs=[pl.BlockSpec((Q, K), lambda: (0, 0))] * 2,
        out_shape=(
            jax.ShapeDtypeStruct((Q, K), jnp.int32),
            jax.ShapeDtypeStruct((Q, K), jnp.float32),
        ),
    )(cv, ci)


def kernel(ray_o, ray_d, points):
    topi, topd2 = _topk_call(ray_o, ray_d, points)

    n = points.shape[0]
    nb = (n + BLK - 1) // BLK
    npad = nb * BLK
    ptsT = jnp.pad(points, ((0, npad - n), (0, 0)),
                   constant_values=1e18).T
    rows3 = _sc_gather(ptsT, topi.reshape(-1))

    qk = Q * K
    rx = rows3[0:qk].reshape(Q, K * CH)
    ry = rows3[qk:2 * qk].reshape(Q, K * CH)
    rz = rows3[2 * qk:].reshape(Q, K * CH)

    full = pl.BlockSpec((Q, K), lambda: (0, 0))
    full3 = pl.BlockSpec((Q, 3), lambda: (0, 0))
    wide = pl.BlockSpec((Q, K * CH), lambda: (0, 0))
    outs = pl.pallas_call(
        _finish_body,
        in_specs=[full3, full3, full, full, wide, wide, wide],
        out_specs=[full] * 7,
        out_shape=(jax.ShapeDtypeStruct((Q, K), jnp.float32),) * 7,
    )(ray_o, ray_d, topd2, topi, rx, ry, rz)
    gx, gy, gz, dist, walk, azim, pit = outs

    ray_info = jnp.concatenate([ray_o, ray_d], axis=-1)
    points_info = jnp.stack([gx, gy, gz, dist, walk, azim, pit], axis=-1)
    return (points, ray_info, points_info, topi)
